# baseline (speedup 1.0000x reference)
import functools

import jax
import jax.numpy as jnp
from jax import lax
from jax.experimental import pallas as pl
from jax.experimental.pallas import tpu as pltpu
from jax.experimental.pallas import tpu_sc as plsc

N = 10000
E = 320000
H = 256
NUM_GRAPHS = 64
BN_EPS = 1e-5

NC = 2
NS = 16
CHUNK = 128
STRIPE = 632
N2 = STRIPE * NS
ZROWS = 64

CPT_A = 160
CPT_B = 80
E3 = CPT_A * NS * CHUNK


IB = 40


def _sc_agg_body(edge_split, table, gidx, dstidx, out, acc, idxs_v, dsts_v,
                 rows0_v, rows1_v, sem0, sem1, ssem0, ssem1):
  core = lax.axis_index("c")
  sid = lax.axis_index("s")

  zeros16 = jnp.zeros((16,), jnp.float32)

  def zbody(i, _):
    r = i // 8
    c = (i % 8) * 16
    rows0_v[r, pl.ds(c, 16)] = zeros16
    return 0

  lax.fori_loop(0, ZROWS * 8, zbody, 0)
  base = sid * STRIPE
  for k in range(10):
    rows = ZROWS if k < 9 else STRIPE - 9 * ZROWS
    pltpu.sync_copy(rows0_v.at[pl.ds(0, rows)],
                    acc.at[pl.ds(base + k * ZROWS, rows)])
  plsc.subcore_barrier()

  cpt = CPT_B if edge_split else CPT_A

  def gather(c, buf, sem):
    return pltpu.async_copy(table.at[idxs_v.at[c]], buf, sem)

  for k in range(cpt // IB):
    sl = pl.ds(k * IB, IB)
    if edge_split:
      wid = sid * NC + core
      pltpu.sync_copy(gidx.at[wid, sl], idxs_v)
      pltpu.sync_copy(dstidx.at[wid, sl], dsts_v)
    else:
      pltpu.sync_copy(gidx.at[core, sid, sl], idxs_v)
      pltpu.sync_copy(dstidx.at[sid, sl], dsts_v)
    gather(0, rows0_v, sem0)
    gather(1, rows1_v, sem1)

    def pbody(p, _):
      c0 = 2 * p
      c1 = c0 + 1
      pltpu.make_async_copy(table.at[idxs_v.at[c0]], rows0_v, sem0).wait()
      pltpu.async_copy(rows0_v, acc.at[dsts_v.at[c0]], ssem0, add=True)
      pltpu.make_async_copy(table.at[idxs_v.at[c1]], rows1_v, sem1).wait()
      pltpu.async_copy(rows1_v, acc.at[dsts_v.at[c1]], ssem1, add=True)
      pltpu.make_async_copy(rows0_v, acc.at[dsts_v.at[c0]], ssem0).wait()

      @pl.when(c0 + 2 < IB)
      def _():
        gather(c0 + 2, rows0_v, sem0)

      pltpu.make_async_copy(rows1_v, acc.at[dsts_v.at[c1]], ssem1).wait()

      @pl.when(c1 + 2 < IB)
      def _():
        gather(c1 + 2, rows1_v, sem1)

      return 0

    lax.fori_loop(0, IB // 2, pbody, 0)
  plsc.subcore_barrier()

  for k in range(10):
    rows = ZROWS if k < 9 else STRIPE - 9 * ZROWS
    sl = pl.ds(base + k * ZROWS, rows)
    pltpu.sync_copy(acc.at[sl], rows0_v.at[pl.ds(0, rows)])
    pltpu.sync_copy(rows0_v.at[pl.ds(0, rows)], out.at[core, sl])


def _sc_agg(table, gidx, dstidx, edge_split):
  def body(table_r, gidx_r, dst_r, out_r, acc_sh, idxs_v, dsts_v, rows0_v,
           rows1_v, sem0, sem1, ssem0, ssem1):
    _sc_agg_body(edge_split, table_r, gidx_r, dst_r, out_r, acc_sh, idxs_v,
                 dsts_v, rows0_v, rows1_v, sem0, sem1, ssem0, ssem1)

  mesh = plsc.VectorSubcoreMesh(core_axis_name="c", subcore_axis_name="s",
                                num_cores=NC, num_subcores=NS)
  f = pl.kernel(
      body,
      out_type=jax.ShapeDtypeStruct((NC, N2, 128), jnp.float32),
      mesh=mesh,
      scratch_types=[
          pltpu.VMEM_SHARED((N2, 128), jnp.float32),
          pltpu.VMEM((IB, CHUNK), jnp.int32),
          pltpu.VMEM((IB, CHUNK), jnp.int32),
          pltpu.VMEM((CHUNK, 128), jnp.float32),
          pltpu.VMEM((CHUNK, 128), jnp.float32),
          pltpu.SemaphoreType.DMA,
          pltpu.SemaphoreType.DMA,
          pltpu.SemaphoreType.DMA,
          pltpu.SemaphoreType.DMA,
      ],
  )
  return f(table, gidx, dstidx)


R = 1000
GSTEPS = N // R


def _mlp_body(concat_agg, h_ref, a0_ref, a1_ref, eps_ref, w1_ref, b1_ref,
              w2_ref, b2_ref, u_ref, s_ref, q_ref):
  i = pl.program_id(0)
  one_eps = 1.0 + eps_ref[0, 0]
  if concat_agg:
    agg = jnp.concatenate([a0_ref[0], a1_ref[0]], axis=1)
  else:
    agg = a0_ref[0] + a1_ref[0]
  v = h_ref[...] * one_eps + agg
  z = jnp.maximum(jnp.dot(v, w1_ref[...],
                          preferred_element_type=jnp.float32) + b1_ref[...], 0.0)
  u = jnp.maximum(jnp.dot(z, w2_ref[...],
                          preferred_element_type=jnp.float32) + b2_ref[...], 0.0)
  u_ref[...] = u
  ps = jnp.sum(u, axis=0, keepdims=True)
  pq = jnp.sum(u * u, axis=0, keepdims=True)

  @pl.when(i == 0)
  def _():
    s_ref[...] = ps
    q_ref[...] = pq

  @pl.when(i > 0)
  def _():
    s_ref[...] += ps
    q_ref[...] += pq


def _mlp(h, agg, eps, w1, b1, w2, b2, concat_agg):
  C = h.shape[1]
  grid = (GSTEPS,)
  return pl.pallas_call(
      functools.partial(_mlp_body, concat_agg),
      grid=grid,
      in_specs=[
          pl.BlockSpec((R, C), lambda i: (i, 0)),
          pl.BlockSpec((1, R, 128), lambda i: (0, i, 0)),
          pl.BlockSpec((1, R, 128), lambda i: (1, i, 0)),
          pl.BlockSpec((1, 1), lambda i: (0, 0)),
          pl.BlockSpec((C, H), lambda i: (0, 0)),
          pl.BlockSpec((1, H), lambda i: (0, 0)),
          pl.BlockSpec((H, H), lambda i: (0, 0)),
          pl.BlockSpec((1, H), lambda i: (0, 0)),
      ],
      out_specs=[
          pl.BlockSpec((R, H), lambda i: (i, 0)),
          pl.BlockSpec((1, H), lambda i: (0, 0)),
          pl.BlockSpec((1, H), lambda i: (0, 0)),
      ],
      out_shape=[
          jax.ShapeDtypeStruct((N, H), jnp.float32),
          jax.ShapeDtypeStruct((1, H), jnp.float32),
          jax.ShapeDtypeStruct((1, H), jnp.float32),
      ],
  )(h, agg, agg, eps, w1, b1, w2, b2)


def _bn_body(u_ref, s_ref, q_ref, g_ref, b_ref, h_ref):
  mu = s_ref[...] * (1.0 / N)
  var = q_ref[...] * (1.0 / N) - mu * mu
  s = g_ref[...] * lax.rsqrt(var + BN_EPS)
  t = b_ref[...] - mu * s
  h_ref[...] = u_ref[...] * s + t


def _bn(u, colsum, colsumsq, gamma, beta):
  return pl.pallas_call(
      _bn_body,
      grid=(GSTEPS,),
      in_specs=[
          pl.BlockSpec((R, H), lambda i: (i, 0)),
          pl.BlockSpec((1, H), lambda i: (0, 0)),
          pl.BlockSpec((1, H), lambda i: (0, 0)),
          pl.BlockSpec((1, H), lambda i: (0, 0)),
          pl.BlockSpec((1, H), lambda i: (0, 0)),
      ],
      out_specs=pl.BlockSpec((R, H), lambda i: (i, 0)),
      out_shape=jax.ShapeDtypeStruct((N, H), jnp.float32),
  )(u, colsum, colsumsq, gamma, beta)


def _head_body(u_ref, b3_ref, s_ref, q_ref, g_ref, be_ref, l1w_ref, l1b_ref,
               l2w_ref, l2b_ref, out_ref, acc_ref, cnt_ref):
  i = pl.program_id(0)
  b = b3_ref[0, 0, :]
  gid = lax.broadcasted_iota(jnp.int32, (R, NUM_GRAPHS), 1)
  oh = (b[:, None] == gid).astype(jnp.float32)
  psum = lax.dot_general(oh, u_ref[...], (((0,), (0,)), ((), ())),
                         preferred_element_type=jnp.float32)
  pcnt = jnp.sum(oh, axis=0, keepdims=True)

  @pl.when(i == 0)
  def _():
    acc_ref[...] = psum
    cnt_ref[...] = pcnt

  @pl.when(i > 0)
  def _():
    acc_ref[...] += psum
    cnt_ref[...] += pcnt

  @pl.when(i == GSTEPS - 1)
  def _():
    mu = s_ref[...] * (1.0 / N)
    var = q_ref[...] * (1.0 / N) - mu * mu
    s = g_ref[...] * lax.rsqrt(var + BN_EPS)
    t = be_ref[...] - mu * s
    cnt = jnp.maximum(cnt_ref[...], 1.0)
    pooled = acc_ref[...] / cnt.reshape(NUM_GRAPHS, 1) * s + t
    y = jnp.maximum(jnp.dot(pooled, l1w_ref[...],
                            preferred_element_type=jnp.float32) + l1b_ref[...],
                    0.0)
    y2 = jnp.dot(y, l2w_ref[...],
                 preferred_element_type=jnp.float32) + l2b_ref[...]
    m = jnp.max(y2, axis=-1, keepdims=True)
    e = y2 - m
    out_ref[...] = e - jnp.log(jnp.sum(jnp.exp(e), axis=-1, keepdims=True))


def _head(u, batch3, colsum, colsumsq, gamma, beta, l1w, l1b, l2w, l2b):
  cout = l2w.shape[1]
  return pl.pallas_call(
      _head_body,
      grid=(GSTEPS,),
      in_specs=[
          pl.BlockSpec((R, H), lambda i: (i, 0)),
          pl.BlockSpec((1, 1, R), lambda i: (i, 0, 0)),
          pl.BlockSpec((1, H), lambda i: (0, 0)),
          pl.BlockSpec((1, H), lambda i: (0, 0)),
          pl.BlockSpec((1, H), lambda i: (0, 0)),
          pl.BlockSpec((1, H), lambda i: (0, 0)),
          pl.BlockSpec((H, H), lambda i: (0, 0)),
          pl.BlockSpec((1, H), lambda i: (0, 0)),
          pl.BlockSpec((H, cout), lambda i: (0, 0)),
          pl.BlockSpec((1, cout), lambda i: (0, 0)),
      ],
      out_specs=pl.BlockSpec((NUM_GRAPHS, cout), lambda i: (0, 0)),
      out_shape=jax.ShapeDtypeStruct((NUM_GRAPHS, cout), jnp.float32),
      scratch_shapes=[
          pltpu.VMEM((NUM_GRAPHS, H), jnp.float32),
          pltpu.VMEM((1, NUM_GRAPHS), jnp.float32),
      ],
  )(u, batch3, colsum, colsumsq, gamma, beta, l1w, l1b, l2w, l2b)


def kernel(x, edge_index, batch, params):
  src = edge_index[0].astype(jnp.int32)
  dst = edge_index[1].astype(jnp.int32)

  npad = E3 - E
  pad_ar = lax.iota(jnp.int32, npad) % 8
  srcp = jnp.concatenate([src, pad_ar])
  dstp = jnp.concatenate([dst, N + pad_ar])

  gidx_a = (2 * srcp)[None, :] + jnp.array([[0], [1]], jnp.int32)
  gidx_a = gidx_a.reshape(NC, NS, CPT_A, CHUNK)
  dst_a = dstp.reshape(NS, CPT_A, CHUNK)
  gidx_b = srcp.reshape(NC * NS, CPT_B, CHUNK)
  dst_b = dstp.reshape(NC * NS, CPT_B, CHUNK)

  eps_of = lambda p: (p["eps"]).reshape(1, 1)
  row = lambda v: v.reshape(1, -1)

  p1 = params["conv1"]
  agg1 = _sc_agg(x, gidx_b, dst_b, edge_split=True)
  u, cs, cq = _mlp(x, agg1, eps_of(p1), p1["w1"], row(p1["b1"]),
                   p1["w2"], row(p1["b2"]), concat_agg=False)
  h = _bn(u, cs, cq, row(p1["bn_g"]), row(p1["bn_b"]))

  for p in params["convs"]:
    table2 = h.reshape(2 * N, 128)
    agg = _sc_agg(table2, gidx_a, dst_a, edge_split=False)
    u, cs, cq = _mlp(h, agg, eps_of(p), p["w1"], row(p["b1"]),
                     p["w2"], row(p["b2"]), concat_agg=True)
    if p is not params["convs"][-1]:
      h = _bn(u, cs, cq, row(p["bn_g"]), row(p["bn_b"]))

  pL = params["convs"][-1]
  batch3 = batch.astype(jnp.int32).reshape(GSTEPS, 1, R)
  return _head(u, batch3, cs, cq, row(pL["bn_g"]), row(pL["bn_b"]),
               params["lin1_w"], row(params["lin1_b"]),
               params["lin2_w"], row(params["lin2_b"]))


# speedup vs baseline: 1.2265x; 1.2265x over previous
import functools

import jax
import jax.numpy as jnp
from jax import lax
from jax.experimental import pallas as pl
from jax.experimental.pallas import tpu as pltpu
from jax.experimental.pallas import tpu_sc as plsc

N = 10000
E = 320000
H = 256
NUM_GRAPHS = 64
BN_EPS = 1e-5

NC = 2
NS = 16
CHUNK = 128
STRIPE = 632
N2 = STRIPE * NS
ZROWS = 64

CPT_A = 160
CPT_B = 80
E3 = CPT_A * NS * CHUNK


IB = 40


def _sc_agg_body(edge_split, table, gidx, dstidx, out, acc, idxs_v, dsts_v,
                 rows0_v, rows1_v, sem0, sem1, ssem0, ssem1):
  core = lax.axis_index("c")
  sid = lax.axis_index("s")

  zeros16 = jnp.zeros((16,), jnp.float32)

  def zbody(i, _):
    r = i // 8
    c = (i % 8) * 16
    rows0_v[r, pl.ds(c, 16)] = zeros16
    return 0

  lax.fori_loop(0, ZROWS * 8, zbody, 0)
  base = sid * STRIPE
  for k in range(10):
    rows = ZROWS if k < 9 else STRIPE - 9 * ZROWS
    pltpu.sync_copy(rows0_v.at[pl.ds(0, rows)],
                    acc.at[pl.ds(base + k * ZROWS, rows)])
  plsc.subcore_barrier()

  cpt = CPT_B if edge_split else CPT_A

  def gather(c, buf, sem):
    return pltpu.async_copy(table.at[idxs_v.at[c]], buf, sem)

  for k in range(cpt // IB):
    sl = pl.ds(k * IB, IB)
    if edge_split:
      wid = sid * NC + core
      pltpu.sync_copy(gidx.at[wid, sl], idxs_v)
      pltpu.sync_copy(dstidx.at[wid, sl], dsts_v)
    else:
      pltpu.sync_copy(gidx.at[core, sid, sl], idxs_v)
      pltpu.sync_copy(dstidx.at[sid, sl], dsts_v)
    gather(0, rows0_v, sem0)

    def pbody(p, _):
      c0 = 2 * p
      c1 = c0 + 1
      gather(c1, rows1_v, sem1)
      pltpu.make_async_copy(table.at[idxs_v.at[c0]], rows0_v, sem0).wait()
      pltpu.sync_copy(rows0_v, acc.at[dsts_v.at[c0]], add=True)

      @pl.when(c1 + 1 < IB)
      def _():
        gather(c1 + 1, rows0_v, sem0)

      pltpu.make_async_copy(table.at[idxs_v.at[c1]], rows1_v, sem1).wait()
      pltpu.sync_copy(rows1_v, acc.at[dsts_v.at[c1]], add=True)
      return 0

    lax.fori_loop(0, IB // 2, pbody, 0)
  plsc.subcore_barrier()

  sl = pl.ds(base, STRIPE)
  pltpu.sync_copy(acc.at[sl], out.at[core, sl])


def _sc_agg(table, gidx, dstidx, edge_split):
  def body(table_r, gidx_r, dst_r, out_r, acc_sh, idxs_v, dsts_v, rows0_v,
           rows1_v, sem0, sem1, ssem0, ssem1):
    _sc_agg_body(edge_split, table_r, gidx_r, dst_r, out_r, acc_sh, idxs_v,
                 dsts_v, rows0_v, rows1_v, sem0, sem1, ssem0, ssem1)

  mesh = plsc.VectorSubcoreMesh(core_axis_name="c", subcore_axis_name="s",
                                num_cores=NC, num_subcores=NS)
  f = pl.kernel(
      body,
      out_type=jax.ShapeDtypeStruct((NC, N2, 128), jnp.float32),
      mesh=mesh,
      scratch_types=[
          pltpu.VMEM_SHARED((N2, 128), jnp.float32),
          pltpu.VMEM((IB, CHUNK), jnp.int32),
          pltpu.VMEM((IB, CHUNK), jnp.int32),
          pltpu.VMEM((CHUNK, 128), jnp.float32),
          pltpu.VMEM((CHUNK, 128), jnp.float32),
          pltpu.SemaphoreType.DMA,
          pltpu.SemaphoreType.DMA,
          pltpu.SemaphoreType.DMA,
          pltpu.SemaphoreType.DMA,
      ],
  )
  return f(table, gidx, dstidx)


R = 1000
GSTEPS = N // R


def _mlp_body(concat_agg, h_ref, a0_ref, a1_ref, eps_ref, w1_ref, b1_ref,
              w2_ref, b2_ref, u_ref, s_ref, q_ref):
  i = pl.program_id(0)
  one_eps = 1.0 + eps_ref[0, 0]
  if concat_agg:
    agg = jnp.concatenate([a0_ref[0], a1_ref[0]], axis=1)
  else:
    agg = a0_ref[0] + a1_ref[0]
  v = h_ref[...] * one_eps + agg
  z = jnp.maximum(jnp.dot(v, w1_ref[...],
                          preferred_element_type=jnp.float32) + b1_ref[...], 0.0)
  u = jnp.maximum(jnp.dot(z, w2_ref[...],
                          preferred_element_type=jnp.float32) + b2_ref[...], 0.0)
  u_ref[...] = u
  ps = jnp.sum(u, axis=0, keepdims=True)
  pq = jnp.sum(u * u, axis=0, keepdims=True)

  @pl.when(i == 0)
  def _():
    s_ref[...] = ps
    q_ref[...] = pq

  @pl.when(i > 0)
  def _():
    s_ref[...] += ps
    q_ref[...] += pq


def _mlp(h, agg, eps, w1, b1, w2, b2, concat_agg):
  C = h.shape[1]
  grid = (GSTEPS,)
  return pl.pallas_call(
      functools.partial(_mlp_body, concat_agg),
      grid=grid,
      in_specs=[
          pl.BlockSpec((R, C), lambda i: (i, 0)),
          pl.BlockSpec((1, R, 128), lambda i: (0, i, 0)),
          pl.BlockSpec((1, R, 128), lambda i: (1, i, 0)),
          pl.BlockSpec((1, 1), lambda i: (0, 0)),
          pl.BlockSpec((C, H), lambda i: (0, 0)),
          pl.BlockSpec((1, H), lambda i: (0, 0)),
          pl.BlockSpec((H, H), lambda i: (0, 0)),
          pl.BlockSpec((1, H), lambda i: (0, 0)),
      ],
      out_specs=[
          pl.BlockSpec((R, H), lambda i: (i, 0)),
          pl.BlockSpec((1, H), lambda i: (0, 0)),
          pl.BlockSpec((1, H), lambda i: (0, 0)),
      ],
      out_shape=[
          jax.ShapeDtypeStruct((N, H), jnp.float32),
          jax.ShapeDtypeStruct((1, H), jnp.float32),
          jax.ShapeDtypeStruct((1, H), jnp.float32),
      ],
  )(h, agg, agg, eps, w1, b1, w2, b2)


def _bn_body(u_ref, s_ref, q_ref, g_ref, b_ref, h_ref):
  mu = s_ref[...] * (1.0 / N)
  var = q_ref[...] * (1.0 / N) - mu * mu
  s = g_ref[...] * lax.rsqrt(var + BN_EPS)
  t = b_ref[...] - mu * s
  h_ref[...] = u_ref[...] * s + t


def _bn(u, colsum, colsumsq, gamma, beta):
  return pl.pallas_call(
      _bn_body,
      grid=(GSTEPS,),
      in_specs=[
          pl.BlockSpec((R, H), lambda i: (i, 0)),
          pl.BlockSpec((1, H), lambda i: (0, 0)),
          pl.BlockSpec((1, H), lambda i: (0, 0)),
          pl.BlockSpec((1, H), lambda i: (0, 0)),
          pl.BlockSpec((1, H), lambda i: (0, 0)),
      ],
      out_specs=pl.BlockSpec((R, H), lambda i: (i, 0)),
      out_shape=jax.ShapeDtypeStruct((N, H), jnp.float32),
  )(u, colsum, colsumsq, gamma, beta)


def _head_body(u_ref, b3_ref, s_ref, q_ref, g_ref, be_ref, l1w_ref, l1b_ref,
               l2w_ref, l2b_ref, out_ref, acc_ref, cnt_ref):
  i = pl.program_id(0)
  b = b3_ref[0, 0, :]
  gid = lax.broadcasted_iota(jnp.int32, (R, NUM_GRAPHS), 1)
  oh = (b[:, None] == gid).astype(jnp.float32)
  psum = lax.dot_general(oh, u_ref[...], (((0,), (0,)), ((), ())),
                         preferred_element_type=jnp.float32)
  pcnt = jnp.sum(oh, axis=0, keepdims=True)

  @pl.when(i == 0)
  def _():
    acc_ref[...] = psum
    cnt_ref[...] = pcnt

  @pl.when(i > 0)
  def _():
    acc_ref[...] += psum
    cnt_ref[...] += pcnt

  @pl.when(i == GSTEPS - 1)
  def _():
    mu = s_ref[...] * (1.0 / N)
    var = q_ref[...] * (1.0 / N) - mu * mu
    s = g_ref[...] * lax.rsqrt(var + BN_EPS)
    t = be_ref[...] - mu * s
    cnt = jnp.maximum(cnt_ref[...], 1.0)
    pooled = acc_ref[...] / cnt.reshape(NUM_GRAPHS, 1) * s + t
    y = jnp.maximum(jnp.dot(pooled, l1w_ref[...],
                            preferred_element_type=jnp.float32) + l1b_ref[...],
                    0.0)
    y2 = jnp.dot(y, l2w_ref[...],
                 preferred_element_type=jnp.float32) + l2b_ref[...]
    m = jnp.max(y2, axis=-1, keepdims=True)
    e = y2 - m
    out_ref[...] = e - jnp.log(jnp.sum(jnp.exp(e), axis=-1, keepdims=True))


def _head(u, batch3, colsum, colsumsq, gamma, beta, l1w, l1b, l2w, l2b):
  cout = l2w.shape[1]
  return pl.pallas_call(
      _head_body,
      grid=(GSTEPS,),
      in_specs=[
          pl.BlockSpec((R, H), lambda i: (i, 0)),
          pl.BlockSpec((1, 1, R), lambda i: (i, 0, 0)),
          pl.BlockSpec((1, H), lambda i: (0, 0)),
          pl.BlockSpec((1, H), lambda i: (0, 0)),
          pl.BlockSpec((1, H), lambda i: (0, 0)),
          pl.BlockSpec((1, H), lambda i: (0, 0)),
          pl.BlockSpec((H, H), lambda i: (0, 0)),
          pl.BlockSpec((1, H), lambda i: (0, 0)),
          pl.BlockSpec((H, cout), lambda i: (0, 0)),
          pl.BlockSpec((1, cout), lambda i: (0, 0)),
      ],
      out_specs=pl.BlockSpec((NUM_GRAPHS, cout), lambda i: (0, 0)),
      out_shape=jax.ShapeDtypeStruct((NUM_GRAPHS, cout), jnp.float32),
      scratch_shapes=[
          pltpu.VMEM((NUM_GRAPHS, H), jnp.float32),
          pltpu.VMEM((1, NUM_GRAPHS), jnp.float32),
      ],
  )(u, batch3, colsum, colsumsq, gamma, beta, l1w, l1b, l2w, l2b)


def kernel(x, edge_index, batch, params):
  src = edge_index[0].astype(jnp.int32)
  dst = edge_index[1].astype(jnp.int32)

  npad = E3 - E
  pad_ar = lax.iota(jnp.int32, npad) % 8
  srcp = jnp.concatenate([src, pad_ar])
  dstp = jnp.concatenate([dst, N + pad_ar])

  gidx_a = (2 * srcp)[None, :] + jnp.array([[0], [1]], jnp.int32)
  gidx_a = gidx_a.reshape(NC, NS, CPT_A, CHUNK)
  dst_a = dstp.reshape(NS, CPT_A, CHUNK)
  gidx_b = srcp.reshape(NC * NS, CPT_B, CHUNK)
  dst_b = dstp.reshape(NC * NS, CPT_B, CHUNK)

  eps_of = lambda p: (p["eps"]).reshape(1, 1)
  row = lambda v: v.reshape(1, -1)

  p1 = params["conv1"]
  agg1 = _sc_agg(x, gidx_b, dst_b, edge_split=True)
  u, cs, cq = _mlp(x, agg1, eps_of(p1), p1["w1"], row(p1["b1"]),
                   p1["w2"], row(p1["b2"]), concat_agg=False)
  h = _bn(u, cs, cq, row(p1["bn_g"]), row(p1["bn_b"]))

  for p in params["convs"]:
    table2 = h.reshape(2 * N, 128)
    agg = _sc_agg(table2, gidx_a, dst_a, edge_split=False)
    u, cs, cq = _mlp(h, agg, eps_of(p), p["w1"], row(p["b1"]),
                     p["w2"], row(p["b2"]), concat_agg=True)
    if p is not params["convs"][-1]:
      h = _bn(u, cs, cq, row(p["bn_g"]), row(p["bn_b"]))

  pL = params["convs"][-1]
  batch3 = batch.astype(jnp.int32).reshape(GSTEPS, 1, R)
  return _head(u, batch3, cs, cq, row(pL["bn_g"]), row(pL["bn_b"]),
               params["lin1_w"], row(params["lin1_b"]),
               params["lin2_w"], row(params["lin2_b"]))
